# Initial kernel scaffold; baseline (speedup 1.0000x reference)
import jax
import jax.numpy as jnp
from jax.experimental import pallas as pl


def kernel(x, edge_index, W, b, alpha):
    raise NotImplementedError("write your pallas kernel here")


# speedup vs baseline: 13.2428x; 13.2428x over previous
import jax
import jax.numpy as jnp
from jax import lax
from jax.experimental import pallas as pl
from jax.experimental.pallas import tpu as pltpu
from jax.experimental.pallas import tpu_sc as plsc

N_NODES = 10000
N_EDGES = 320000
D = 128
H = D // 2

NC = 2
NS = 16
NPAD = 10240
EPAD = 327680
EROWS = EPAD // 128
ROWS_PER_TILE = EROWS // (NC * NS)
STRIPE = NPAD // NS
DEG_W = 16


def _deg_body(dst_hbm, out_hbm, idx_v, ones_v, zbuf_v, deg_sh):
    c = lax.axis_index("c")
    s = lax.axis_index("s")
    gwid = c * NS + s
    ones16 = jnp.full((16,), 1.0, jnp.float32)
    zeros16 = jnp.zeros((16,), jnp.float32)
    for j in range(128):
        ones_v[j] = ones16
        zbuf_v[j] = zeros16
    for k in range(STRIPE // 128):
        pltpu.sync_copy(zbuf_v, deg_sh.at[pl.ds(s * STRIPE + k * 128, 128), :])
    pltpu.sync_copy(dst_hbm.at[pl.ds(gwid * ROWS_PER_TILE, ROWS_PER_TILE), :],
                    idx_v)
    plsc.subcore_barrier()
    for j in range(ROWS_PER_TILE):
        pltpu.sync_copy(ones_v, deg_sh.at[idx_v.at[j]], add=True)
    plsc.subcore_barrier()
    pltpu.sync_copy(deg_sh.at[pl.ds(s * STRIPE, STRIPE), :],
                    out_hbm.at[c, pl.ds(s * STRIPE, STRIPE), :])


def _deg_sc(dst):
    mesh = plsc.VectorSubcoreMesh(core_axis_name="c", subcore_axis_name="s")
    f = pl.kernel(
        _deg_body,
        compiler_params=pltpu.CompilerParams(use_tc_tiling_on_sc=False),
        out_type=jax.ShapeDtypeStruct((NC, NPAD, DEG_W), jnp.float32),
        mesh=mesh,
        scratch_types=[
            pltpu.VMEM((ROWS_PER_TILE, 128), jnp.int32),
            pltpu.VMEM((128, DEG_W), jnp.float32),
            pltpu.VMEM((128, DEG_W), jnp.float32),
            pltpu.VMEM_SHARED((NPAD, DEG_W), jnp.float32),
        ],
    )
    return f(dst)


NBUF = 4


def _edge_body(y0_hbm, y1_hbm, src_hbm, dst_hbm, out_hbm,
               srcidx_v, dstidx_v, b0, b1, b2, b3, zbuf_v, acc_sh,
               sem_g, sem_s):
    bufs = (b0, b1, b2, b3)
    c = lax.axis_index("c")
    s = lax.axis_index("s")
    gwid = c * NS + s
    zeros16 = jnp.zeros((16,), jnp.float32)
    for r in range(32):
        for q in range(H // 16):
            zbuf_v[r, pl.ds(q * 16, 16)] = zeros16
    base = gwid * ROWS_PER_TILE
    pltpu.sync_copy(src_hbm.at[pl.ds(base, ROWS_PER_TILE), :], srcidx_v)
    pltpu.sync_copy(dst_hbm.at[pl.ds(base, ROWS_PER_TILE), :], dstidx_v)

    for h, y_hbm in enumerate((y0_hbm, y1_hbm)):
        for k in range(STRIPE // 32):
            pltpu.sync_copy(zbuf_v,
                            acc_sh.at[pl.ds(s * STRIPE + k * 32, 32), :])
        plsc.subcore_barrier()

        def one_pass(p, carry):
            descs_g = []
            for b in range(NBUF):
                g = p * NBUF + b
                descs_g.append(
                    pltpu.async_copy(y_hbm.at[srcidx_v.at[g]], bufs[b],
                                     sem_g))
            for dsc in descs_g:
                dsc.wait()
            descs_s = []
            for b in range(NBUF):
                g = p * NBUF + b
                descs_s.append(
                    pltpu.async_copy(bufs[b], acc_sh.at[dstidx_v.at[g]],
                                     sem_s, add=True))
            for dsc in descs_s:
                dsc.wait()
            return carry

        lax.fori_loop(0, ROWS_PER_TILE // NBUF, one_pass, 0)
        plsc.subcore_barrier()
        pltpu.sync_copy(acc_sh.at[pl.ds(s * STRIPE, STRIPE), :],
                        out_hbm.at[c, h, pl.ds(s * STRIPE, STRIPE), :])
        plsc.subcore_barrier()


def _edges_sc(y0, y1, src, dst):
    mesh = plsc.VectorSubcoreMesh(core_axis_name="c", subcore_axis_name="s")
    f = pl.kernel(
        _edge_body,
        compiler_params=pltpu.CompilerParams(use_tc_tiling_on_sc=False),
        out_type=jax.ShapeDtypeStruct((NC, 2, NPAD, H), jnp.float32),
        mesh=mesh,
        scratch_types=[
            pltpu.VMEM((ROWS_PER_TILE, 128), jnp.int32),
            pltpu.VMEM((ROWS_PER_TILE, 128), jnp.int32),
            pltpu.VMEM((128, H), jnp.float32),
            pltpu.VMEM((128, H), jnp.float32),
            pltpu.VMEM((128, H), jnp.float32),
            pltpu.VMEM((128, H), jnp.float32),
            pltpu.VMEM((32, H), jnp.float32),
            pltpu.VMEM_SHARED((NPAD, H), jnp.float32),
            pltpu.SemaphoreType.DMA,
            pltpu.SemaphoreType.DMA,
        ],
    )
    return f(y0, y1, src, dst)


BLK = 2048


def _y_body(x_ref, w_ref, degp_ref, y0_ref, y1_ref):
    xw = jnp.dot(x_ref[...], w_ref[...], preferred_element_type=jnp.float32)
    deg = degp_ref[0, :, 0:1] + degp_ref[1, :, 0:1] + 1.0
    y = xw * lax.rsqrt(deg)
    y0_ref[...] = y[:, :H]
    y1_ref[...] = y[:, H:]


def _y_tc(x_pad, W, degp):
    return pl.pallas_call(
        _y_body,
        grid=(NPAD // BLK,),
        in_specs=[
            pl.BlockSpec((BLK, D), lambda i: (i, 0)),
            pl.BlockSpec((D, D), lambda i: (0, 0)),
            pl.BlockSpec((NC, BLK, DEG_W), lambda i: (0, i, 0)),
        ],
        out_specs=[
            pl.BlockSpec((BLK, H), lambda i: (i, 0)),
            pl.BlockSpec((BLK, H), lambda i: (i, 0)),
        ],
        out_shape=[
            jax.ShapeDtypeStruct((NPAD, H), jnp.float32),
            jax.ShapeDtypeStruct((NPAD, H), jnp.float32),
        ],
    )(x_pad, W, degp)


def _comb_body(p_ref, y0_ref, y1_ref, degp_ref, b_ref, a_ref, o_ref):
    deg = degp_ref[0, :, 0:1] + degp_ref[1, :, 0:1] + 1.0
    dinv = lax.rsqrt(deg)
    for h, y_ref in enumerate((y0_ref, y1_ref)):
        o = dinv * (p_ref[0, h] + p_ref[1, h] + y_ref[...]) \
            + b_ref[0, pl.ds(h * H, H)]
        o_ref[:, pl.ds(h * H, H)] = jnp.where(
            o > 0, o, a_ref[0, pl.ds(h * H, H)] * o)


def _comb_tc(parts, y0, y1, degp, b2, a2):
    return pl.pallas_call(
        _comb_body,
        grid=(NPAD // BLK,),
        in_specs=[
            pl.BlockSpec((NC, 2, BLK, H), lambda i: (0, 0, i, 0)),
            pl.BlockSpec((BLK, H), lambda i: (i, 0)),
            pl.BlockSpec((BLK, H), lambda i: (i, 0)),
            pl.BlockSpec((NC, BLK, DEG_W), lambda i: (0, i, 0)),
            pl.BlockSpec((1, D), lambda i: (0, 0)),
            pl.BlockSpec((1, D), lambda i: (0, 0)),
        ],
        out_specs=pl.BlockSpec((BLK, D), lambda i: (i, 0)),
        out_shape=jax.ShapeDtypeStruct((NPAD, D), jnp.float32),
    )(parts, y0, y1, degp, b2, a2)


def kernel(x, edge_index, W, b, alpha):
    ei = edge_index.astype(jnp.int32)
    pad = jnp.full((EPAD - N_EDGES,), N_NODES, jnp.int32)
    src = jnp.concatenate([ei[0], pad]).reshape(EROWS, 128)
    dst = jnp.concatenate([ei[1], pad]).reshape(EROWS, 128)
    x_pad = jnp.pad(x, ((0, NPAD - N_NODES), (0, 0)))

    degp = _deg_sc(dst)
    y0, y1 = _y_tc(x_pad, W, degp)
    parts = _edges_sc(y0, y1, src, dst)
    o = _comb_tc(parts, y0, y1, degp, b.reshape(1, D), alpha.reshape(1, D))
    return o[:N_NODES]


# speedup vs baseline: 14.0365x; 1.0599x over previous
import jax
import jax.numpy as jnp
from jax import lax
from jax.experimental import pallas as pl
from jax.experimental.pallas import tpu as pltpu
from jax.experimental.pallas import tpu_sc as plsc

N_NODES = 10000
N_EDGES = 320000
D = 128
H = D // 2

NC = 2
NS = 16
NPAD = 10240
EPAD = 327680
EROWS = EPAD // 128
ROWS_PER_TILE = EROWS // (NC * NS)
STRIPE = NPAD // NS
DEG_W = 16


def _deg_body(dst_hbm, out_hbm, idx_v, ones_v, zbuf_v, deg_sh):
    c = lax.axis_index("c")
    s = lax.axis_index("s")
    gwid = c * NS + s
    ones16 = jnp.full((16,), 1.0, jnp.float32)
    zeros16 = jnp.zeros((16,), jnp.float32)
    for j in range(128):
        ones_v[j] = ones16
        zbuf_v[j] = zeros16
    for k in range(STRIPE // 128):
        pltpu.sync_copy(zbuf_v, deg_sh.at[pl.ds(s * STRIPE + k * 128, 128), :])
    pltpu.sync_copy(dst_hbm.at[pl.ds(gwid * ROWS_PER_TILE, ROWS_PER_TILE), :],
                    idx_v)
    plsc.subcore_barrier()
    for j in range(ROWS_PER_TILE):
        pltpu.sync_copy(ones_v, deg_sh.at[idx_v.at[j]], add=True)
    plsc.subcore_barrier()
    pltpu.sync_copy(deg_sh.at[pl.ds(s * STRIPE, STRIPE), :],
                    out_hbm.at[c, pl.ds(s * STRIPE, STRIPE), :])


def _deg_sc(dst):
    mesh = plsc.VectorSubcoreMesh(core_axis_name="c", subcore_axis_name="s")
    f = pl.kernel(
        _deg_body,
        compiler_params=pltpu.CompilerParams(use_tc_tiling_on_sc=False),
        out_type=jax.ShapeDtypeStruct((NC, NPAD, DEG_W), jnp.float32),
        mesh=mesh,
        scratch_types=[
            pltpu.VMEM((ROWS_PER_TILE, 128), jnp.int32),
            pltpu.VMEM((128, DEG_W), jnp.float32),
            pltpu.VMEM((128, DEG_W), jnp.float32),
            pltpu.VMEM_SHARED((NPAD, DEG_W), jnp.float32),
        ],
    )
    return f(dst)


NBUF = 8
NSETS = ROWS_PER_TILE // NBUF


def _edge_body(y0_hbm, y1_hbm, src_hbm, dst_hbm, out_hbm,
               srcidx_v, dstidx_v, b0, b1, b2, b3, b4, b5, b6, b7,
               zbuf_v, acc_sh, sem_g0, sem_g1, sem_s0, sem_s1):
    grp = ((b0, b1, b2, b3), (b4, b5, b6, b7))
    sem_g = (sem_g0, sem_g1)
    sem_s = (sem_s0, sem_s1)
    c = lax.axis_index("c")
    s = lax.axis_index("s")
    gwid = c * NS + s
    zeros16 = jnp.zeros((16,), jnp.float32)
    for r in range(32):
        for q in range(H // 16):
            zbuf_v[r, pl.ds(q * 16, 16)] = zeros16
    base = gwid * ROWS_PER_TILE
    pltpu.sync_copy(src_hbm.at[pl.ds(base, ROWS_PER_TILE), :], srcidx_v)
    pltpu.sync_copy(dst_hbm.at[pl.ds(base, ROWS_PER_TILE), :], dstidx_v)

    for h, y_hbm in enumerate((y0_hbm, y1_hbm)):
        for k in range(STRIPE // 32):
            pltpu.sync_copy(zbuf_v,
                            acc_sh.at[pl.ds(s * STRIPE + k * 32, 32), :])
        plsc.subcore_barrier()

        def fire_gathers(dp, grp_i):
            ds_ = []
            for b in range(4):
                g = dp * NBUF + grp_i * 4 + b
                ds_.append(pltpu.async_copy(y_hbm.at[srcidx_v.at[g]],
                                            grp[grp_i][b], sem_g[grp_i]))
            return ds_

        def drain_fire_scatters(dp, grp_i, gds):
            for dsc in gds:
                dsc.wait()
            ds_ = []
            for b in range(4):
                g = dp * NBUF + grp_i * 4 + b
                ds_.append(pltpu.async_copy(grp[grp_i][b],
                                            acc_sh.at[dstidx_v.at[g]],
                                            sem_s[grp_i], add=True))
            return ds_

        def gather_descs(dp, grp_i):
            return [pltpu.make_async_copy(
                y_hbm.at[srcidx_v.at[dp * NBUF + grp_i * 4 + b]],
                grp[grp_i][b], sem_g[grp_i]) for b in range(4)]

        def body(dp, refire):
            sA = drain_fire_scatters(dp, 0, gather_descs(dp, 0))
            sB = drain_fire_scatters(dp, 1, gather_descs(dp, 1))
            if refire:
                for dsc in sA:
                    dsc.wait()
                fire_gathers(dp + 1, 0)
                for dsc in sB:
                    dsc.wait()
                fire_gathers(dp + 1, 1)
            else:
                for dsc in sA:
                    dsc.wait()
                for dsc in sB:
                    dsc.wait()

        fire_gathers(0, 0)
        fire_gathers(0, 1)

        def loop_body(dp, carry):
            body(dp, refire=True)
            return carry

        if NSETS > 1:
            lax.fori_loop(0, NSETS - 1, loop_body, 0)
        body(NSETS - 1, refire=False)

        plsc.subcore_barrier()
        pltpu.sync_copy(acc_sh.at[pl.ds(s * STRIPE, STRIPE), :],
                        out_hbm.at[c, h, pl.ds(s * STRIPE, STRIPE), :])
        plsc.subcore_barrier()


def _edges_sc(y0, y1, src, dst):
    mesh = plsc.VectorSubcoreMesh(core_axis_name="c", subcore_axis_name="s")
    f = pl.kernel(
        _edge_body,
        compiler_params=pltpu.CompilerParams(use_tc_tiling_on_sc=False),
        out_type=jax.ShapeDtypeStruct((NC, 2, NPAD, H), jnp.float32),
        mesh=mesh,
        scratch_types=(
            [pltpu.VMEM((ROWS_PER_TILE, 128), jnp.int32)] * 2
            + [pltpu.VMEM((128, H), jnp.float32)] * NBUF
            + [pltpu.VMEM((32, H), jnp.float32),
               pltpu.VMEM_SHARED((NPAD, H), jnp.float32)]
            + [pltpu.SemaphoreType.DMA] * 4
        ),
    )
    return f(y0, y1, src, dst)


BLK = 2048


def _y_body(x_ref, w_ref, degp_ref, y0_ref, y1_ref):
    xw = jnp.dot(x_ref[...], w_ref[...], preferred_element_type=jnp.float32)
    deg = degp_ref[0, :, 0:1] + degp_ref[1, :, 0:1] + 1.0
    y = xw * lax.rsqrt(deg)
    y0_ref[...] = y[:, :H]
    y1_ref[...] = y[:, H:]


def _y_tc(x_pad, W, degp):
    return pl.pallas_call(
        _y_body,
        grid=(NPAD // BLK,),
        in_specs=[
            pl.BlockSpec((BLK, D), lambda i: (i, 0)),
            pl.BlockSpec((D, D), lambda i: (0, 0)),
            pl.BlockSpec((NC, BLK, DEG_W), lambda i: (0, i, 0)),
        ],
        out_specs=[
            pl.BlockSpec((BLK, H), lambda i: (i, 0)),
            pl.BlockSpec((BLK, H), lambda i: (i, 0)),
        ],
        out_shape=[
            jax.ShapeDtypeStruct((NPAD, H), jnp.float32),
            jax.ShapeDtypeStruct((NPAD, H), jnp.float32),
        ],
    )(x_pad, W, degp)


def _comb_body(p_ref, y0_ref, y1_ref, degp_ref, b_ref, a_ref, o_ref):
    deg = degp_ref[0, :, 0:1] + degp_ref[1, :, 0:1] + 1.0
    dinv = lax.rsqrt(deg)
    for h, y_ref in enumerate((y0_ref, y1_ref)):
        o = dinv * (p_ref[0, h] + p_ref[1, h] + y_ref[...]) \
            + b_ref[0, pl.ds(h * H, H)]
        o_ref[:, pl.ds(h * H, H)] = jnp.where(
            o > 0, o, a_ref[0, pl.ds(h * H, H)] * o)


def _comb_tc(parts, y0, y1, degp, b2, a2):
    return pl.pallas_call(
        _comb_body,
        grid=(NPAD // BLK,),
        in_specs=[
            pl.BlockSpec((NC, 2, BLK, H), lambda i: (0, 0, i, 0)),
            pl.BlockSpec((BLK, H), lambda i: (i, 0)),
            pl.BlockSpec((BLK, H), lambda i: (i, 0)),
            pl.BlockSpec((NC, BLK, DEG_W), lambda i: (0, i, 0)),
            pl.BlockSpec((1, D), lambda i: (0, 0)),
            pl.BlockSpec((1, D), lambda i: (0, 0)),
        ],
        out_specs=pl.BlockSpec((BLK, D), lambda i: (i, 0)),
        out_shape=jax.ShapeDtypeStruct((NPAD, D), jnp.float32),
    )(parts, y0, y1, degp, b2, a2)


def kernel(x, edge_index, W, b, alpha):
    ei = edge_index.astype(jnp.int32)
    pad = jnp.full((EPAD - N_EDGES,), N_NODES, jnp.int32)
    src = jnp.concatenate([ei[0], pad]).reshape(EROWS, 128)
    dst = jnp.concatenate([ei[1], pad]).reshape(EROWS, 128)
    x_pad = jnp.pad(x, ((0, NPAD - N_NODES), (0, 0)))

    degp = _deg_sc(dst)
    y0, y1 = _y_tc(x_pad, W, degp)
    parts = _edges_sc(y0, y1, src, dst)
    o = _comb_tc(parts, y0, y1, degp, b.reshape(1, D), alpha.reshape(1, D))
    return o[:N_NODES]


# speedup vs baseline: 30.3487x; 2.1621x over previous
import jax
import jax.numpy as jnp
from jax import lax
from jax.experimental import pallas as pl
from jax.experimental.pallas import tpu as pltpu
from jax.experimental.pallas import tpu_sc as plsc

N_NODES = 10000
N_EDGES = 320000
D = 128
H = D // 2

NC = 2
NS = 16
NPAD = 10240
EPAD = 327680
EROWS = EPAD // 128
ROWS_PER_TILE = EROWS // (NC * NS)
STRIPE = NPAD // NS
DEG_W = 16


def _deg_body(dst_hbm, out_hbm, idx_v, ones_v, zbuf_v, deg_sh):
    c = lax.axis_index("c")
    s = lax.axis_index("s")
    gwid = c * NS + s
    ones16 = jnp.full((16,), 1.0, jnp.float32)
    zeros16 = jnp.zeros((16,), jnp.float32)
    for j in range(128):
        ones_v[j] = ones16
        zbuf_v[j] = zeros16
    for k in range(STRIPE // 128):
        pltpu.sync_copy(zbuf_v, deg_sh.at[pl.ds(s * STRIPE + k * 128, 128), :])
    pltpu.sync_copy(dst_hbm.at[pl.ds(gwid * ROWS_PER_TILE, ROWS_PER_TILE), :],
                    idx_v)
    plsc.subcore_barrier()
    for j in range(ROWS_PER_TILE):
        pltpu.sync_copy(ones_v, deg_sh.at[idx_v.at[j]], add=True)
    plsc.subcore_barrier()
    pltpu.sync_copy(deg_sh.at[pl.ds(s * STRIPE, STRIPE), :],
                    out_hbm.at[c, pl.ds(s * STRIPE, STRIPE), :])


def _deg_sc(dst):
    mesh = plsc.VectorSubcoreMesh(core_axis_name="c", subcore_axis_name="s")
    f = pl.kernel(
        _deg_body,
        compiler_params=pltpu.CompilerParams(use_tc_tiling_on_sc=False),
        out_type=jax.ShapeDtypeStruct((NC, NPAD, DEG_W), jnp.float32),
        mesh=mesh,
        scratch_types=[
            pltpu.VMEM((ROWS_PER_TILE, 128), jnp.int32),
            pltpu.VMEM((128, DEG_W), jnp.float32),
            pltpu.VMEM((128, DEG_W), jnp.float32),
            pltpu.VMEM_SHARED((NPAD, DEG_W), jnp.float32),
        ],
    )
    return f(dst)


NBUF = 8
NSETS = ROWS_PER_TILE // NBUF


def _edge_body(y0_hbm, y1_hbm, src_hbm, dst_hbm, out_hbm,
               srcidx_v, dstidx_v, b0, b1, b2, b3, b4, b5, b6, b7,
               zbuf_v, acc_sh, sem_g0, sem_g1, sem_s0, sem_s1):
    grp = ((b0, b1, b2, b3), (b4, b5, b6, b7))
    sem_g = (sem_g0, sem_g1)
    sem_s = (sem_s0, sem_s1)
    c = lax.axis_index("c")
    s = lax.axis_index("s")
    gwid = c * NS + s
    zeros16 = jnp.zeros((16,), jnp.float32)
    for r in range(32):
        for q in range(H // 16):
            zbuf_v[r, pl.ds(q * 16, 16)] = zeros16
    base = gwid * ROWS_PER_TILE
    pltpu.sync_copy(src_hbm.at[pl.ds(base, ROWS_PER_TILE), :], srcidx_v)
    pltpu.sync_copy(dst_hbm.at[pl.ds(base, ROWS_PER_TILE), :], dstidx_v)

    for h, y_hbm in enumerate((y0_hbm, y1_hbm)):
        for k in range(STRIPE // 32):
            pltpu.sync_copy(zbuf_v,
                            acc_sh.at[pl.ds(s * STRIPE + k * 32, 32), :])
        plsc.subcore_barrier()

        def fire_gathers(dp, grp_i):
            ds_ = []
            for b in range(4):
                g = dp * NBUF + grp_i * 4 + b
                ds_.append(pltpu.async_copy(y_hbm.at[srcidx_v.at[g]],
                                            grp[grp_i][b], sem_g[grp_i]))
            return ds_

        def drain_fire_scatters(dp, grp_i, gds):
            for dsc in gds:
                dsc.wait()
            ds_ = []
            for b in range(4):
                g = dp * NBUF + grp_i * 4 + b
                ds_.append(pltpu.async_copy(grp[grp_i][b],
                                            acc_sh.at[dstidx_v.at[g]],
                                            sem_s[grp_i], add=True))
            return ds_

        def gather_descs(dp, grp_i):
            return [pltpu.make_async_copy(
                y_hbm.at[srcidx_v.at[dp * NBUF + grp_i * 4 + b]],
                grp[grp_i][b], sem_g[grp_i]) for b in range(4)]

        def body(dp, refire):
            sA = drain_fire_scatters(dp, 0, gather_descs(dp, 0))
            sB = drain_fire_scatters(dp, 1, gather_descs(dp, 1))
            if refire:
                for dsc in sA:
                    dsc.wait()
                fire_gathers(dp + 1, 0)
                for dsc in sB:
                    dsc.wait()
                fire_gathers(dp + 1, 1)
            else:
                for dsc in sA:
                    dsc.wait()
                for dsc in sB:
                    dsc.wait()

        fire_gathers(0, 0)
        fire_gathers(0, 1)

        def loop_body(dp, carry):
            body(dp, refire=True)
            return carry

        if NSETS > 1:
            lax.fori_loop(0, NSETS - 1, loop_body, 0)
        body(NSETS - 1, refire=False)

        plsc.subcore_barrier()
        pltpu.sync_copy(acc_sh.at[pl.ds(s * STRIPE, STRIPE), :],
                        out_hbm.at[c, h, pl.ds(s * STRIPE, STRIPE), :])
        plsc.subcore_barrier()


def _edges_sc(y0, y1, src, dst):
    mesh = plsc.VectorSubcoreMesh(core_axis_name="c", subcore_axis_name="s")
    f = pl.kernel(
        _edge_body,
        compiler_params=pltpu.CompilerParams(use_tc_tiling_on_sc=False),
        out_type=jax.ShapeDtypeStruct((NC, 2, NPAD, H), jnp.float32),
        mesh=mesh,
        scratch_types=(
            [pltpu.VMEM((ROWS_PER_TILE, 128), jnp.int32)] * 2
            + [pltpu.VMEM((128, H), jnp.float32)] * NBUF
            + [pltpu.VMEM((32, H), jnp.float32),
               pltpu.VMEM_SHARED((NPAD, H), jnp.float32)]
            + [pltpu.SemaphoreType.DMA] * 4
        ),
    )
    return f(y0, y1, src, dst)


BLK = 2048


def _y_body(x_ref, w_ref, degp_ref, y0_ref, y1_ref):
    xw = jnp.dot(x_ref[...], w_ref[...], preferred_element_type=jnp.float32)
    deg = degp_ref[0, :, 0:1] + degp_ref[1, :, 0:1] + 1.0
    y = xw * lax.rsqrt(deg)
    y0_ref[...] = y[:, :H]
    y1_ref[...] = y[:, H:]


def _y_tc(x_pad, W, degp):
    return pl.pallas_call(
        _y_body,
        grid=(NPAD // BLK,),
        in_specs=[
            pl.BlockSpec((BLK, D), lambda i: (i, 0)),
            pl.BlockSpec((D, D), lambda i: (0, 0)),
            pl.BlockSpec((NC, BLK, DEG_W), lambda i: (0, i, 0)),
        ],
        out_specs=[
            pl.BlockSpec((BLK, H), lambda i: (i, 0)),
            pl.BlockSpec((BLK, H), lambda i: (i, 0)),
        ],
        out_shape=[
            jax.ShapeDtypeStruct((NPAD, H), jnp.float32),
            jax.ShapeDtypeStruct((NPAD, H), jnp.float32),
        ],
    )(x_pad, W, degp)


def _comb_body(p_ref, y0_ref, y1_ref, degp_ref, b_ref, a_ref, o_ref):
    deg = degp_ref[0, :, 0:1] + degp_ref[1, :, 0:1] + 1.0
    dinv = lax.rsqrt(deg)
    for h, y_ref in enumerate((y0_ref, y1_ref)):
        o = dinv * (p_ref[0, h] + p_ref[1, h] + y_ref[...]) \
            + b_ref[0, pl.ds(h * H, H)]
        o_ref[:, pl.ds(h * H, H)] = jnp.where(
            o > 0, o, a_ref[0, pl.ds(h * H, H)] * o)


def _comb_tc(parts, y0, y1, degp, b2, a2):
    return pl.pallas_call(
        _comb_body,
        grid=(NPAD // BLK,),
        in_specs=[
            pl.BlockSpec((NC, 2, BLK, H), lambda i: (0, 0, i, 0)),
            pl.BlockSpec((BLK, H), lambda i: (i, 0)),
            pl.BlockSpec((BLK, H), lambda i: (i, 0)),
            pl.BlockSpec((NC, BLK, DEG_W), lambda i: (0, i, 0)),
            pl.BlockSpec((1, D), lambda i: (0, 0)),
            pl.BlockSpec((1, D), lambda i: (0, 0)),
        ],
        out_specs=pl.BlockSpec((BLK, D), lambda i: (i, 0)),
        out_shape=jax.ShapeDtypeStruct((NPAD, D), jnp.float32),
    )(parts, y0, y1, degp, b2, a2)


def kernel(x, edge_index, W, b, alpha):
    ei = edge_index.astype(jnp.int32)
    pad = N_NODES + jnp.arange(EPAD - N_EDGES, dtype=jnp.int32) \
        % (NPAD - N_NODES)
    src = jnp.concatenate([ei[0], pad]).reshape(EROWS, 128)
    dst = jnp.concatenate([ei[1], pad]).reshape(EROWS, 128)
    x_pad = jnp.pad(x, ((0, NPAD - N_NODES), (0, 0)))

    degp = _deg_sc(dst)
    y0, y1 = _y_tc(x_pad, W, degp)
    parts = _edges_sc(y0, y1, src, dst)
    o = _comb_tc(parts, y0, y1, degp, b.reshape(1, D), alpha.reshape(1, D))
    return o[:N_NODES]
